# Initial kernel scaffold; baseline (speedup 1.0000x reference)
import jax
import jax.numpy as jnp
from jax.experimental import pallas as pl


def kernel(acoustic_seq, visual_seq, IS_BAG_list, hW, hb, wW, wb, convW, convb):
    raise NotImplementedError("write your pallas kernel here")


# speedup vs baseline: 14.3695x; 14.3695x over previous
import jax
import jax.numpy as jnp
from jax import lax
from jax.experimental import pallas as pl
from jax.experimental.pallas import tpu as pltpu

_L = 2048
_T = 32
_D = 32
_G = _L // _T


def _body(conv_ref, hW_ref, hb_ref, wWT_ref, wb_ref, a_ref, v_ref,
          oa_ref, ov_ref):
    a = a_ref[0]
    v = v_ref[0]
    row_g = lax.broadcasted_iota(jnp.int32, (_L, _T), 0) // _G
    col_g = lax.broadcasted_iota(jnp.int32, (_L, _T), 1)
    P = (row_g == col_g).astype(jnp.float32)
    dn_red = (((0,), (0,)), ((), ()))
    A_r = lax.dot_general(P, a, dn_red) * (1.0 / _G)
    V_r = lax.dot_general(P, v, dn_red) * (1.0 / _G)
    w0 = conv_ref[0]
    w1 = conv_ref[1]
    cb = conv_ref[2]
    c = jax.nn.sigmoid(w0 * A_r + w1 * V_r + cb)
    hw = (A_r + V_r) * 0.5
    rm = jnp.mean(hw, axis=1, keepdims=True)
    cm = jnp.mean(hw, axis=0, keepdims=True)
    h = jax.nn.sigmoid(jnp.dot(hW_ref[...], rm) + hb_ref[...])
    w = jax.nn.sigmoid(jnp.dot(cm, wWT_ref[...]) + wb_ref[...])
    S = (h + w + c) * (1.0 / 3.0)
    dn_up = (((1,), (0,)), ((), ()))
    scale = lax.dot_general(P, S, dn_up)
    oa_ref[0] = a * scale
    ov_ref[0] = v * scale


def kernel(acoustic_seq, visual_seq, IS_BAG_list, hW, hb, wW, wb, convW,
           convb):
    del IS_BAG_list
    B = acoustic_seq.shape[0]
    conv = jnp.stack([convW[0, 0, 0, 0], convW[0, 1, 0, 0], convb[0]])
    hb2 = hb.reshape(_T, 1)
    wb2 = wb.reshape(1, _D)
    wWT = wW.T
    seq_spec = pl.BlockSpec((1, _L, _D), lambda i: (i, 0, 0))
    full = lambda *s: pl.BlockSpec(s, lambda i: tuple(0 for _ in s))
    out_a, out_v = pl.pallas_call(
        _body,
        grid=(B,),
        in_specs=[
            pl.BlockSpec(memory_space=pltpu.SMEM),
            full(_T, _T),
            full(_T, 1),
            full(_D, _D),
            full(1, _D),
            seq_spec,
            seq_spec,
        ],
        out_specs=[seq_spec, seq_spec],
        out_shape=[
            jax.ShapeDtypeStruct((B, _L, _D), jnp.float32),
            jax.ShapeDtypeStruct((B, _L, _D), jnp.float32),
        ],
    )(conv, hW, hb2, wWT, wb2, acoustic_seq, visual_seq)
    return out_a, out_v
